# baseline (speedup 1.0000x reference)
import jax
import jax.numpy as jnp
from jax import lax
from jax.experimental import pallas as pl
from jax.experimental.pallas import tpu as pltpu

_TM = 2048


def _router_block(x_ref, w_ref, b_ref, gates_ref, idx_ref):
    logits = lax.dot_general(
        x_ref[...],
        w_ref[...],
        ((( 1,), (1,)), ((), ())),
        preferred_element_type=jnp.float32,
    ) + b_ref[...]

    tm, ne = logits.shape
    ef = lax.broadcasted_iota(jnp.int32, (tm, ne), 1).astype(jnp.float32)

    m1 = jnp.max(logits, axis=1, keepdims=True)
    i1 = jnp.min(jnp.where(logits == m1, ef, float(ne)), axis=1, keepdims=True)
    sel1 = ef == i1

    masked = jnp.where(sel1, -jnp.inf, logits)
    m2 = jnp.max(masked, axis=1, keepdims=True)
    i2 = jnp.min(jnp.where(masked == m2, ef, float(ne)), axis=1, keepdims=True)
    sel2 = ef == i2

    t = jnp.exp(m2 - m1)
    denom = 1.0 + t
    g1 = 1.0 / denom
    g2 = t / denom

    gates_ref[...] = jnp.where(sel1, g1, 0.0) + jnp.where(sel2, g2, 0.0)
    idx_ref[...] = jnp.concatenate([i1, i2], axis=1).astype(jnp.int32)


def kernel(x, gate_W, gate_b):
    n_tokens, d_model = x.shape
    n_experts = gate_W.shape[0]
    b2 = gate_b.reshape(1, n_experts)

    grid = (n_tokens // _TM,)
    gates, idx = pl.pallas_call(
        _router_block,
        grid=grid,
        in_specs=[
            pl.BlockSpec((_TM, d_model), lambda i: (i, 0)),
            pl.BlockSpec((n_experts, d_model), lambda i: (0, 0)),
            pl.BlockSpec((1, n_experts), lambda i: (0, 0)),
        ],
        out_specs=[
            pl.BlockSpec((_TM, n_experts), lambda i: (i, 0)),
            pl.BlockSpec((_TM, 2), lambda i: (i, 0)),
        ],
        out_shape=[
            jax.ShapeDtypeStruct((n_tokens, n_experts), jnp.float32),
            jax.ShapeDtypeStruct((n_tokens, 2), jnp.int32),
        ],
    )(x, gate_W, b2)
    return gates, idx


# speedup vs baseline: 1.0347x; 1.0347x over previous
import jax
import jax.numpy as jnp
from jax import lax
from jax.experimental import pallas as pl
from jax.experimental.pallas import tpu as pltpu

_TM = 4096


def _router_block(x_ref, w_ref, b_ref, gates_ref, idx_ref):
    logits = lax.dot_general(
        x_ref[...],
        w_ref[...],
        ((( 1,), (1,)), ((), ())),
        preferred_element_type=jnp.float32,
    ) + b_ref[...]

    tm, ne = logits.shape
    ef = lax.broadcasted_iota(jnp.int32, (tm, ne), 1).astype(jnp.float32)

    m1 = jnp.max(logits, axis=1, keepdims=True)
    i1 = jnp.min(jnp.where(logits == m1, ef, float(ne)), axis=1, keepdims=True)
    sel1 = ef == i1

    masked = jnp.where(sel1, -jnp.inf, logits)
    m2 = jnp.max(masked, axis=1, keepdims=True)
    i2 = jnp.min(jnp.where(masked == m2, ef, float(ne)), axis=1, keepdims=True)
    sel2 = ef == i2

    t = jnp.exp(m2 - m1)
    denom = 1.0 + t
    g1 = 1.0 / denom
    g2 = t / denom

    gates_ref[...] = jnp.where(sel1, g1, 0.0) + jnp.where(sel2, g2, 0.0)
    idx_ref[...] = jnp.concatenate([i1, i2], axis=1).astype(jnp.int32)


def kernel(x, gate_W, gate_b):
    n_tokens, d_model = x.shape
    n_experts = gate_W.shape[0]
    b2 = gate_b.reshape(1, n_experts)

    grid = (n_tokens // _TM,)
    gates, idx = pl.pallas_call(
        _router_block,
        grid=grid,
        in_specs=[
            pl.BlockSpec((_TM, d_model), lambda i: (i, 0)),
            pl.BlockSpec((n_experts, d_model), lambda i: (0, 0)),
            pl.BlockSpec((1, n_experts), lambda i: (0, 0)),
        ],
        out_specs=[
            pl.BlockSpec((_TM, n_experts), lambda i: (i, 0)),
            pl.BlockSpec((_TM, 2), lambda i: (i, 0)),
        ],
        out_shape=[
            jax.ShapeDtypeStruct((n_tokens, n_experts), jnp.float32),
            jax.ShapeDtypeStruct((n_tokens, 2), jnp.int32),
        ],
    )(x, gate_W, b2)
    return gates, idx
